# Initial kernel scaffold; baseline (speedup 1.0000x reference)
import jax
import jax.numpy as jnp
from jax.experimental import pallas as pl


def kernel(stab_xy, syndrome, dist_emb, S):
    raise NotImplementedError("write your pallas kernel here")


# speedup vs baseline: 2.1025x; 2.1025x over previous
import functools

import jax
import jax.numpy as jnp
from jax.experimental import pallas as pl

_BS = 64


def _bias_kernel(row_x_ref, row_y_ref, col_x_ref, col_y_ref, tab_ref, out_ref):
    b = out_ref.shape[0]
    rx = row_x_ref[...][:, :, None]
    ry = row_y_ref[...][:, :, None]
    cx = col_x_ref[...][None, :, :]
    cy = col_y_ref[...][None, :, :]
    dist = jnp.abs(rx - cx) + jnp.abs(ry - cy)
    dist = jnp.minimum(dist, 8.0)
    acc = jnp.broadcast_to(tab_ref[0, :][None, None, :], dist.shape)
    for d in range(1, 9):
        acc = jnp.where(dist == float(d), tab_ref[d, :][None, None, :], acc)
    out_ref[...] = jnp.broadcast_to(acc[None], (b,) + acc.shape)


@functools.partial(jax.jit, static_argnums=(3,))
def _run(stab_xy, syndrome, dist_emb, S):
    B = syndrome.shape[0]
    DB = dist_emb.shape[1]
    xy = stab_xy.astype(jnp.float32)
    row_x = xy[:, 0:1]
    row_y = xy[:, 1:2]
    col_of_lane = jnp.arange(128, dtype=jnp.int32) // DB
    col_idx = 8 * jnp.arange(S // 8, dtype=jnp.int32)[:, None] + col_of_lane[None, :]
    col_x = xy[col_idx, 0]
    col_y = xy[col_idx, 1]
    tab = jnp.tile(dist_emb, (1, 128 // DB))

    grid = (S // _BS,)
    out = pl.pallas_call(
        _bias_kernel,
        grid=grid,
        in_specs=[
            pl.BlockSpec((_BS, 1), lambda i: (i, 0)),
            pl.BlockSpec((_BS, 1), lambda i: (i, 0)),
            pl.BlockSpec((S // 8, 128), lambda i: (0, 0)),
            pl.BlockSpec((S // 8, 128), lambda i: (0, 0)),
            pl.BlockSpec((9, 128), lambda i: (0, 0)),
        ],
        out_specs=pl.BlockSpec((B, _BS, S // 8, 128), lambda i: (0, i, 0, 0)),
        out_shape=jax.ShapeDtypeStruct((B, S, S // 8, 128), jnp.float32),
    )(row_x, row_y, col_x, col_y, tab)
    return out.reshape(B, S, S, DB)


def kernel(stab_xy, syndrome, dist_emb, S):
    return _run(stab_xy, syndrome, dist_emb, stab_xy.shape[0])
